# Initial kernel scaffold; baseline (speedup 1.0000x reference)
import functools
import math

import jax
import jax.numpy as jnp
from jax import lax
from jax.experimental import pallas as pl
from jax.experimental.pallas import tpu as pltpu
from jax.experimental.pallas import tpu_sc as plsc

N = 10000
E = 320000
H = 128
G = 50
F = 128
CUTOFF = 10.0
SHIFT = math.log(2.0)

NC = 2
NS = 16
NW = NC * NS
CH = 128
NCHUNK = E // CH
MAXJ = (NCHUNK + NW - 1) // NW
CW = 16

NZFULL = N // CH
NZREM = N - NZFULL * CH


def _ssp(t):
    return jnp.log1p(jnp.exp(-jnp.abs(t))) + jnp.maximum(t, 0.0) - SHIFT


def _lin1_body(x_ref, w_ref, o_ref):
    o_ref[...] = lax.dot_general(
        x_ref[...], w_ref[...], (((1,), (1,)), ((), ())),
        preferred_element_type=jnp.float32)


def _lin1(x, w):
    bn = 1000
    return pl.pallas_call(
        _lin1_body,
        grid=(N // bn,),
        in_specs=[pl.BlockSpec((bn, H), lambda i: (i, 0)),
                  pl.BlockSpec((F, H), lambda i: (0, 0))],
        out_specs=pl.BlockSpec((bn, F), lambda i: (i, 0)),
        out_shape=jax.ShapeDtypeStruct((N, F), jnp.float32),
    )(x, w)


_MESH = plsc.VectorSubcoreMesh(
    core_axis_name="c", subcore_axis_name="s", num_cores=NC, num_subcores=NS)


@functools.partial(
    pl.kernel,
    out_type=jax.ShapeDtypeStruct((NCHUNK, CH, F), jnp.float32),
    mesh=_MESH,
    scratch_types=[
        pltpu.VMEM((1, CH), jnp.int32),
        pltpu.VMEM((CH, F), jnp.float32),
        pltpu.SemaphoreType.DMA,
    ],
)
def _gather_k(h_hbm, src_hbm, out_hbm, idx_v, rows_v, sem):
    wid = lax.axis_index("s") * NC + lax.axis_index("c")

    @pl.loop(0, MAXJ)
    def _chunks(j):
        c = wid + NW * j

        @pl.when(c < NCHUNK)
        def _():
            pltpu.sync_copy(src_hbm.at[c], idx_v.at[0])
            pltpu.async_copy(h_hbm.at[idx_v.at[0]], rows_v, sem).wait()
            pltpu.sync_copy(rows_v, out_hbm.at[c])


def _edge_body(ea_ref, ew_ref, h_ref, w1_ref, b1_ref, w2_ref, b2_ref, o_ref):
    a = lax.dot_general(ea_ref[...], w1_ref[...], (((1,), (1,)), ((), ())),
                        preferred_element_type=jnp.float32) + b1_ref[...]
    a = _ssp(a)
    wf = lax.dot_general(a, w2_ref[...], (((1,), (1,)), ((), ())),
                         preferred_element_type=jnp.float32) + b2_ref[...]
    cc = 0.5 * (jnp.cos(ew_ref[...] * (math.pi / CUTOFF)) + 1.0)
    o_ref[...] = wf * cc * h_ref[...]


def _edge_mlp(edge_attr, ew2, hsrc, w1, b1, w2, b2):
    eb = 1280
    return pl.pallas_call(
        _edge_body,
        grid=(E // eb,),
        in_specs=[pl.BlockSpec((eb, G), lambda i: (i, 0)),
                  pl.BlockSpec((eb, 1), lambda i: (i, 0)),
                  pl.BlockSpec((eb, F), lambda i: (i, 0)),
                  pl.BlockSpec((F, G), lambda i: (0, 0)),
                  pl.BlockSpec((1, F), lambda i: (0, 0)),
                  pl.BlockSpec((F, F), lambda i: (0, 0)),
                  pl.BlockSpec((1, F), lambda i: (0, 0))],
        out_specs=pl.BlockSpec((eb, F), lambda i: (i, 0)),
        out_shape=jax.ShapeDtypeStruct((E, F), jnp.float32),
    )(edge_attr, ew2, hsrc, w1, b1, w2, b2)


@functools.partial(
    pl.kernel,
    out_type=(jax.ShapeDtypeStruct((NC, N, F), jnp.float32),
              jax.ShapeDtypeStruct((NC, N, CW), jnp.float32)),
    mesh=_MESH,
    scratch_types=[
        pltpu.VMEM((1, CH), jnp.int32),
        pltpu.VMEM((CH, F), jnp.float32),
        pltpu.VMEM((CH, CW), jnp.float32),
        pltpu.VMEM_SHARED((N, F), jnp.float32),
        pltpu.VMEM_SHARED((N, CW), jnp.float32),
    ],
)
def _scatter_k(msg_hbm, dst_hbm, zrow_hbm, zcnt_hbm, ones_hbm,
               ssum_hbm, cnt_hbm, idx_v, rows_v, ones_v, acc_s, cacc_s):
    cid = lax.axis_index("c")
    sid = lax.axis_index("s")
    wid = sid * NC + cid

    pltpu.sync_copy(ones_hbm, ones_v)

    @pl.loop(0, NZFULL)
    def _z(z):
        @pl.when(lax.rem(z, NS) == sid)
        def _():
            pltpu.sync_copy(zrow_hbm, acc_s.at[pl.ds(z * CH, CH), :])
            pltpu.sync_copy(zcnt_hbm, cacc_s.at[pl.ds(z * CH, CH), :])

    @pl.when(sid == NS - 1)
    def _zrem():
        pltpu.sync_copy(zrow_hbm.at[pl.ds(0, NZREM), :],
                        acc_s.at[pl.ds(NZFULL * CH, NZREM), :])
        pltpu.sync_copy(zcnt_hbm.at[pl.ds(0, NZREM), :],
                        cacc_s.at[pl.ds(NZFULL * CH, NZREM), :])

    plsc.subcore_barrier()

    @pl.loop(0, MAXJ)
    def _chunks(j):
        c = wid + NW * j

        @pl.when(c < NCHUNK)
        def _():
            pltpu.sync_copy(dst_hbm.at[c], idx_v.at[0])
            pltpu.sync_copy(msg_hbm.at[c], rows_v)
            pltpu.sync_copy(rows_v, acc_s.at[idx_v.at[0]], add=True)
            pltpu.sync_copy(ones_v, cacc_s.at[idx_v.at[0]], add=True)

    plsc.subcore_barrier()

    @pl.loop(0, NZFULL)
    def _o(z):
        @pl.when(lax.rem(z, NS) == sid)
        def _():
            pltpu.sync_copy(acc_s.at[pl.ds(z * CH, CH), :],
                            ssum_hbm.at[cid, pl.ds(z * CH, CH), :])
            pltpu.sync_copy(cacc_s.at[pl.ds(z * CH, CH), :],
                            cnt_hbm.at[cid, pl.ds(z * CH, CH), :])

    @pl.when(sid == NS - 1)
    def _orem():
        pltpu.sync_copy(acc_s.at[pl.ds(NZFULL * CH, NZREM), :],
                        ssum_hbm.at[cid, pl.ds(NZFULL * CH, NZREM), :])
        pltpu.sync_copy(cacc_s.at[pl.ds(NZFULL * CH, NZREM), :],
                        cnt_hbm.at[cid, pl.ds(NZFULL * CH, NZREM), :])


def _final_body(s_ref, c_ref, w2_ref, b2_ref, w_ref, b_ref, o_ref):
    s = s_ref[0] + s_ref[1]
    cnt = c_ref[0, :, 0:1] + c_ref[1, :, 0:1]
    mean = s / jnp.maximum(cnt, 1.0)
    t = lax.dot_general(mean, w2_ref[...], (((1,), (1,)), ((), ())),
                        preferred_element_type=jnp.float32) + b2_ref[...]
    t = _ssp(t)
    o_ref[...] = lax.dot_general(t, w_ref[...], (((1,), (1,)), ((), ())),
                                 preferred_element_type=jnp.float32) + b_ref[...]


def _final(ssum, cnt, lin2_w, lin2_b, lin_w, lin_b):
    bn = 1000
    return pl.pallas_call(
        _final_body,
        grid=(N // bn,),
        in_specs=[pl.BlockSpec((NC, bn, F), lambda i: (0, i, 0)),
                  pl.BlockSpec((NC, bn, CW), lambda i: (0, i, 0)),
                  pl.BlockSpec((H, F), lambda i: (0, 0)),
                  pl.BlockSpec((1, H), lambda i: (0, 0)),
                  pl.BlockSpec((H, H), lambda i: (0, 0)),
                  pl.BlockSpec((1, H), lambda i: (0, 0))],
        out_specs=pl.BlockSpec((bn, H), lambda i: (i, 0)),
        out_shape=jax.ShapeDtypeStruct((N, H), jnp.float32),
    )(ssum, cnt, lin2_w, lin2_b, lin_w, lin_b)


def kernel(x, edge_index, edge_weight, edge_attr, mlp_w1, mlp_b1, mlp_w2,
           mlp_b2, lin1_w, lin2_w, lin2_b, lin_w, lin_b):
    h = _lin1(x, lin1_w)
    src2 = edge_index[0].reshape(NCHUNK, CH)
    dst2 = edge_index[1].reshape(NCHUNK, CH)
    hsrc = _gather_k(h, src2)
    msg = _edge_mlp(edge_attr, edge_weight.reshape(E, 1),
                    hsrc.reshape(E, F), mlp_w1, mlp_b1.reshape(1, F),
                    mlp_w2, mlp_b2.reshape(1, F))
    zrow = jnp.zeros((CH, F), jnp.float32)
    zcnt = jnp.zeros((CH, CW), jnp.float32)
    ones = jnp.ones((CH, CW), jnp.float32)
    ssum, cnt = _scatter_k(msg.reshape(NCHUNK, CH, F), dst2, zrow, zcnt, ones)
    return _final(ssum, cnt, lin2_w, lin2_b.reshape(1, H),
                  lin_w, lin_b.reshape(1, H))


# speedup vs baseline: 1.9218x; 1.9218x over previous
import functools
import math

import jax
import jax.numpy as jnp
from jax import lax
from jax.experimental import pallas as pl
from jax.experimental.pallas import tpu as pltpu
from jax.experimental.pallas import tpu_sc as plsc

N = 10000
E = 320000
H = 128
G = 50
F = 128
CUTOFF = 10.0
SHIFT = math.log(2.0)

NC = 2
NS = 16
NW = NC * NS
CH = 128
NCHUNK = E // CH
MAXJ = (NCHUNK + NW - 1) // NW

NZFULL = N // CH
NZREM = N - NZFULL * CH


def _ssp(t):
    return jnp.log1p(jnp.exp(-jnp.abs(t))) + jnp.maximum(t, 0.0) - SHIFT


def _lin1_body(x_ref, w_ref, o_ref):
    o_ref[...] = lax.dot_general(
        x_ref[...], w_ref[...], (((1,), (1,)), ((), ())),
        preferred_element_type=jnp.float32)


def _lin1(x, w):
    bn = 1000
    return pl.pallas_call(
        _lin1_body,
        grid=(N // bn,),
        in_specs=[pl.BlockSpec((bn, H), lambda i: (i, 0)),
                  pl.BlockSpec((F, H), lambda i: (0, 0))],
        out_specs=pl.BlockSpec((bn, F), lambda i: (i, 0)),
        out_shape=jax.ShapeDtypeStruct((N, F), jnp.float32),
    )(x, w)


_MESH = plsc.VectorSubcoreMesh(
    core_axis_name="c", subcore_axis_name="s", num_cores=NC, num_subcores=NS)


@functools.partial(
    pl.kernel,
    out_type=jax.ShapeDtypeStruct((NCHUNK, CH, F), jnp.float32),
    mesh=_MESH,
    scratch_types=[
        pltpu.VMEM((CH,), jnp.int32),
        pltpu.VMEM((CH, F), jnp.float32),
        pltpu.SemaphoreType.DMA,
    ],
)
def _gather_k(h_hbm, src_hbm, out_hbm, idx_v, rows_v, sem):
    wid = lax.axis_index("s") * NC + lax.axis_index("c")

    @pl.loop(0, MAXJ)
    def _chunks(j):
        c = wid + NW * j

        @pl.when(c < NCHUNK)
        def _():
            pltpu.sync_copy(src_hbm.at[c], idx_v)
            pltpu.async_copy(h_hbm.at[idx_v], rows_v, sem).wait()
            pltpu.sync_copy(rows_v, out_hbm.at[c])


def _edge_body(ea_ref, ew_ref, h_ref, w1_ref, b1_ref, w2_ref, b2_ref, o_ref):
    a = lax.dot_general(ea_ref[...], w1_ref[...], (((1,), (1,)), ((), ())),
                        preferred_element_type=jnp.float32) + b1_ref[...]
    a = _ssp(a)
    wf = lax.dot_general(a, w2_ref[...], (((1,), (1,)), ((), ())),
                         preferred_element_type=jnp.float32) + b2_ref[...]
    cc = 0.5 * (jnp.cos(ew_ref[...] * (math.pi / CUTOFF)) + 1.0)
    o_ref[...] = wf * cc * h_ref[...]


def _edge_mlp(edge_attr, ew2, hsrc, w1, b1, w2, b2):
    eb = 1280
    return pl.pallas_call(
        _edge_body,
        grid=(E // eb,),
        in_specs=[pl.BlockSpec((eb, G), lambda i: (i, 0)),
                  pl.BlockSpec((eb, 1), lambda i: (i, 0)),
                  pl.BlockSpec((eb, F), lambda i: (i, 0)),
                  pl.BlockSpec((F, G), lambda i: (0, 0)),
                  pl.BlockSpec((1, F), lambda i: (0, 0)),
                  pl.BlockSpec((F, F), lambda i: (0, 0)),
                  pl.BlockSpec((1, F), lambda i: (0, 0))],
        out_specs=pl.BlockSpec((eb, F), lambda i: (i, 0)),
        out_shape=jax.ShapeDtypeStruct((E, F), jnp.float32),
    )(edge_attr, ew2, hsrc, w1, b1, w2, b2)


@functools.partial(
    pl.kernel,
    out_type=jax.ShapeDtypeStruct((NC, N, F), jnp.float32),
    mesh=_MESH,
    scratch_types=[
        pltpu.VMEM((CH,), jnp.int32),
        pltpu.VMEM((CH, F), jnp.float32),
        pltpu.VMEM_SHARED((N, F), jnp.float32),
    ],
)
def _scatter_k(msg_hbm, dst_hbm, zrow_hbm, ssum_hbm, idx_v, rows_v, acc_s):
    cid = lax.axis_index("c")
    sid = lax.axis_index("s")
    wid = sid * NC + cid

    pltpu.sync_copy(zrow_hbm, rows_v)

    @pl.loop(0, NZFULL)
    def _z(z):
        @pl.when(lax.rem(z, NS) == sid)
        def _():
            pltpu.sync_copy(rows_v, acc_s.at[pl.ds(z * CH, CH), :])

    @pl.when(sid == NS - 1)
    def _zrem():
        pltpu.sync_copy(rows_v.at[pl.ds(0, NZREM), :],
                        acc_s.at[pl.ds(NZFULL * CH, NZREM), :])

    plsc.subcore_barrier()

    @pl.loop(0, MAXJ)
    def _chunks(j):
        c = wid + NW * j

        @pl.when(c < NCHUNK)
        def _():
            pltpu.sync_copy(dst_hbm.at[c], idx_v)
            pltpu.sync_copy(msg_hbm.at[c], rows_v)
            pltpu.sync_copy(rows_v, acc_s.at[idx_v], add=True)

    plsc.subcore_barrier()

    @pl.loop(0, NZFULL)
    def _o(z):
        @pl.when(lax.rem(z, NS) == sid)
        def _():
            pltpu.sync_copy(acc_s.at[pl.ds(z * CH, CH), :], rows_v)
            pltpu.sync_copy(rows_v, ssum_hbm.at[cid, pl.ds(z * CH, CH), :])

    @pl.when(sid == NS - 1)
    def _orem():
        pltpu.sync_copy(acc_s.at[pl.ds(NZFULL * CH, NZREM), :],
                        rows_v.at[pl.ds(0, NZREM), :])
        pltpu.sync_copy(rows_v.at[pl.ds(0, NZREM), :],
                        ssum_hbm.at[cid, pl.ds(NZFULL * CH, NZREM), :])


@functools.partial(
    pl.kernel,
    out_type=jax.ShapeDtypeStruct((NC, N, F), jnp.float32),
    mesh=_MESH,
    scratch_types=[
        pltpu.VMEM((CH,), jnp.int32),
        pltpu.VMEM((CH, F), jnp.float32),
        pltpu.VMEM_SHARED((N, F), jnp.float32),
    ],
)
def _count_k(dst_hbm, zrow_hbm, ones_hbm, cnt_hbm, idx_v, rows_v, acc_s):
    cid = lax.axis_index("c")
    sid = lax.axis_index("s")
    wid = sid * NC + cid

    pltpu.sync_copy(zrow_hbm, rows_v)

    @pl.loop(0, NZFULL)
    def _z(z):
        @pl.when(lax.rem(z, NS) == sid)
        def _():
            pltpu.sync_copy(rows_v, acc_s.at[pl.ds(z * CH, CH), :])

    @pl.when(sid == NS - 1)
    def _zrem():
        pltpu.sync_copy(rows_v.at[pl.ds(0, NZREM), :],
                        acc_s.at[pl.ds(NZFULL * CH, NZREM), :])

    pltpu.sync_copy(ones_hbm, rows_v)
    plsc.subcore_barrier()

    @pl.loop(0, MAXJ)
    def _chunks(j):
        c = wid + NW * j

        @pl.when(c < NCHUNK)
        def _():
            pltpu.sync_copy(dst_hbm.at[c], idx_v)
            pltpu.sync_copy(rows_v, acc_s.at[idx_v], add=True)

    plsc.subcore_barrier()

    @pl.loop(0, NZFULL)
    def _o(z):
        @pl.when(lax.rem(z, NS) == sid)
        def _():
            pltpu.sync_copy(acc_s.at[pl.ds(z * CH, CH), :], rows_v)
            pltpu.sync_copy(rows_v, cnt_hbm.at[cid, pl.ds(z * CH, CH), :])

    @pl.when(sid == NS - 1)
    def _orem():
        pltpu.sync_copy(acc_s.at[pl.ds(NZFULL * CH, NZREM), :],
                        rows_v.at[pl.ds(0, NZREM), :])
        pltpu.sync_copy(rows_v.at[pl.ds(0, NZREM), :],
                        cnt_hbm.at[cid, pl.ds(NZFULL * CH, NZREM), :])


def _final_body(s_ref, c_ref, w2_ref, b2_ref, w_ref, b_ref, o_ref):
    s = s_ref[0] + s_ref[1]
    cnt = c_ref[0, :, 0:1] + c_ref[1, :, 0:1]
    mean = s / jnp.maximum(cnt, 1.0)
    t = lax.dot_general(mean, w2_ref[...], (((1,), (1,)), ((), ())),
                        preferred_element_type=jnp.float32) + b2_ref[...]
    t = _ssp(t)
    o_ref[...] = lax.dot_general(t, w_ref[...], (((1,), (1,)), ((), ())),
                                 preferred_element_type=jnp.float32) + b_ref[...]


def _final(ssum, cnt, lin2_w, lin2_b, lin_w, lin_b):
    bn = 1000
    return pl.pallas_call(
        _final_body,
        grid=(N // bn,),
        in_specs=[pl.BlockSpec((NC, bn, F), lambda i: (0, i, 0)),
                  pl.BlockSpec((NC, bn, F), lambda i: (0, i, 0)),
                  pl.BlockSpec((H, F), lambda i: (0, 0)),
                  pl.BlockSpec((1, H), lambda i: (0, 0)),
                  pl.BlockSpec((H, H), lambda i: (0, 0)),
                  pl.BlockSpec((1, H), lambda i: (0, 0))],
        out_specs=pl.BlockSpec((bn, H), lambda i: (i, 0)),
        out_shape=jax.ShapeDtypeStruct((N, H), jnp.float32),
    )(ssum, cnt, lin2_w, lin2_b, lin_w, lin_b)


def kernel(x, edge_index, edge_weight, edge_attr, mlp_w1, mlp_b1, mlp_w2,
           mlp_b2, lin1_w, lin2_w, lin2_b, lin_w, lin_b):
    h = _lin1(x, lin1_w)
    src2 = edge_index[0].reshape(NCHUNK, CH)
    dst2 = edge_index[1].reshape(NCHUNK, CH)
    zrow = jnp.zeros((CH, F), jnp.float32)
    ones = jnp.ones((CH, F), jnp.float32)
    hsrc = _gather_k(h, src2)
    cnt = _count_k(dst2, zrow, ones)
    msg = _edge_mlp(edge_attr, edge_weight.reshape(E, 1),
                    hsrc.reshape(E, F), mlp_w1, mlp_b1.reshape(1, F),
                    mlp_w2, mlp_b2.reshape(1, F))
    ssum = _scatter_k(msg.reshape(NCHUNK, CH, F), dst2, zrow)
    return _final(ssum, cnt, lin2_w, lin2_b.reshape(1, H),
                  lin_w, lin_b.reshape(1, H))


# speedup vs baseline: 2.6686x; 1.3885x over previous
import functools
import math

import jax
import jax.numpy as jnp
from jax import lax
from jax.experimental import pallas as pl
from jax.experimental.pallas import tpu as pltpu
from jax.experimental.pallas import tpu_sc as plsc

N = 10000
E = 320000
H = 128
G = 50
F = 128
CUTOFF = 10.0
SHIFT = math.log(2.0)

NC = 2
NS = 16
NW = NC * NS
CH = 128
NCHUNK = E // CH
MAXJ = (NCHUNK + NW - 1) // NW

NZFULL = N // CH
NZREM = N - NZFULL * CH


def _ssp(t):
    return jnp.log1p(jnp.exp(-jnp.abs(t))) + jnp.maximum(t, 0.0) - SHIFT


def _lin1_body(x_ref, w_ref, o_ref):
    o_ref[...] = lax.dot_general(
        x_ref[...], w_ref[...], (((1,), (1,)), ((), ())),
        preferred_element_type=jnp.float32)


def _lin1(x, w):
    bn = 1000
    return pl.pallas_call(
        _lin1_body,
        grid=(N // bn,),
        in_specs=[pl.BlockSpec((bn, H), lambda i: (i, 0)),
                  pl.BlockSpec((F, H), lambda i: (0, 0))],
        out_specs=pl.BlockSpec((bn, F), lambda i: (i, 0)),
        out_shape=jax.ShapeDtypeStruct((N, F), jnp.float32),
    )(x, w)


_MESH = plsc.VectorSubcoreMesh(
    core_axis_name="c", subcore_axis_name="s", num_cores=NC, num_subcores=NS)


@functools.partial(
    pl.kernel,
    out_type=jax.ShapeDtypeStruct((NCHUNK, CH, F), jnp.float32),
    mesh=_MESH,
    scratch_types=[
        pltpu.VMEM((CH,), jnp.int32),
        pltpu.VMEM((CH, F), jnp.float32),
        pltpu.SemaphoreType.DMA,
    ],
)
def _gather_k(h_hbm, src_hbm, out_hbm, idx_v, rows_v, sem):
    wid = lax.axis_index("s") * NC + lax.axis_index("c")

    @pl.loop(0, MAXJ)
    def _chunks(j):
        c = wid + NW * j

        @pl.when(c < NCHUNK)
        def _():
            pltpu.sync_copy(src_hbm.at[c], idx_v)
            pltpu.async_copy(h_hbm.at[idx_v], rows_v, sem).wait()
            pltpu.sync_copy(rows_v, out_hbm.at[c])


def _cw_body(ew_ref, o_ref):
    o_ref[...] = 0.5 * (jnp.cos(ew_ref[...] * (math.pi / CUTOFF)) + 1.0)


def _cutoff(ew2d):
    return pl.pallas_call(
        _cw_body,
        grid=(1,),
        in_specs=[pl.BlockSpec((NCHUNK, CH), lambda i: (0, 0))],
        out_specs=pl.BlockSpec((NCHUNK, CH), lambda i: (0, 0)),
        out_shape=jax.ShapeDtypeStruct((NCHUNK, CH), jnp.float32),
    )(ew2d)


def _edge_body(ea_ref, cc_ref, h_ref, w1_ref, b1_ref, w2_ref, b2_ref, o_ref):
    a = lax.dot_general(ea_ref[...], w1_ref[...], (((1,), (1,)), ((), ())),
                        preferred_element_type=jnp.float32) + b1_ref[...]
    a = _ssp(a)
    wf = lax.dot_general(a, w2_ref[...], (((1,), (1,)), ((), ())),
                         preferred_element_type=jnp.float32) + b2_ref[...]
    o_ref[...] = wf * cc_ref[...] * h_ref[...]


def _edge_mlp(edge_attr, cc2, hsrc, w1, b1, w2, b2):
    eb = 1280
    return pl.pallas_call(
        _edge_body,
        grid=(E // eb,),
        in_specs=[pl.BlockSpec((eb, G), lambda i: (i, 0)),
                  pl.BlockSpec((eb, 1), lambda i: (i, 0)),
                  pl.BlockSpec((eb, F), lambda i: (i, 0)),
                  pl.BlockSpec((F, G), lambda i: (0, 0)),
                  pl.BlockSpec((1, F), lambda i: (0, 0)),
                  pl.BlockSpec((F, F), lambda i: (0, 0)),
                  pl.BlockSpec((1, F), lambda i: (0, 0))],
        out_specs=pl.BlockSpec((eb, F), lambda i: (i, 0)),
        out_shape=jax.ShapeDtypeStruct((E, F), jnp.float32),
    )(edge_attr, cc2, hsrc, w1, b1, w2, b2)


@functools.partial(
    pl.kernel,
    out_type=jax.ShapeDtypeStruct((NC, N, F), jnp.float32),
    mesh=_MESH,
    scratch_types=[
        pltpu.VMEM((CH,), jnp.int32),
        pltpu.VMEM((CH, F), jnp.float32),
        pltpu.VMEM_SHARED((N, F), jnp.float32),
    ],
)
def _scatter_k(msg_hbm, dst_hbm, zrow_hbm, ssum_hbm, idx_v, rows_v, acc_s):
    cid = lax.axis_index("c")
    sid = lax.axis_index("s")
    wid = sid * NC + cid

    pltpu.sync_copy(zrow_hbm, rows_v)

    @pl.loop(0, NZFULL)
    def _z(z):
        @pl.when(lax.rem(z, NS) == sid)
        def _():
            pltpu.sync_copy(rows_v, acc_s.at[pl.ds(z * CH, CH), :])

    @pl.when(sid == NS - 1)
    def _zrem():
        pltpu.sync_copy(rows_v.at[pl.ds(0, NZREM), :],
                        acc_s.at[pl.ds(NZFULL * CH, NZREM), :])

    plsc.subcore_barrier()

    @pl.loop(0, MAXJ)
    def _chunks(j):
        c = wid + NW * j

        @pl.when(c < NCHUNK)
        def _():
            pltpu.sync_copy(dst_hbm.at[c], idx_v)
            pltpu.sync_copy(msg_hbm.at[c], rows_v)
            pltpu.sync_copy(rows_v, acc_s.at[idx_v], add=True)

    plsc.subcore_barrier()

    @pl.loop(0, NZFULL)
    def _o(z):
        @pl.when(lax.rem(z, NS) == sid)
        def _():
            pltpu.sync_copy(acc_s.at[pl.ds(z * CH, CH), :], rows_v)
            pltpu.sync_copy(rows_v, ssum_hbm.at[cid, pl.ds(z * CH, CH), :])

    @pl.when(sid == NS - 1)
    def _orem():
        pltpu.sync_copy(acc_s.at[pl.ds(NZFULL * CH, NZREM), :],
                        rows_v.at[pl.ds(0, NZREM), :])
        pltpu.sync_copy(rows_v.at[pl.ds(0, NZREM), :],
                        ssum_hbm.at[cid, pl.ds(NZFULL * CH, NZREM), :])


@functools.partial(
    pl.kernel,
    out_type=jax.ShapeDtypeStruct((NC, N, F), jnp.float32),
    mesh=_MESH,
    scratch_types=[
        pltpu.VMEM((CH,), jnp.int32),
        pltpu.VMEM((CH, F), jnp.float32),
        pltpu.VMEM_SHARED((N, F), jnp.float32),
    ],
)
def _count_k(dst_hbm, zrow_hbm, ones_hbm, cnt_hbm, idx_v, rows_v, acc_s):
    cid = lax.axis_index("c")
    sid = lax.axis_index("s")
    wid = sid * NC + cid

    pltpu.sync_copy(zrow_hbm, rows_v)

    @pl.loop(0, NZFULL)
    def _z(z):
        @pl.when(lax.rem(z, NS) == sid)
        def _():
            pltpu.sync_copy(rows_v, acc_s.at[pl.ds(z * CH, CH), :])

    @pl.when(sid == NS - 1)
    def _zrem():
        pltpu.sync_copy(rows_v.at[pl.ds(0, NZREM), :],
                        acc_s.at[pl.ds(NZFULL * CH, NZREM), :])

    pltpu.sync_copy(ones_hbm, rows_v)
    plsc.subcore_barrier()

    @pl.loop(0, MAXJ)
    def _chunks(j):
        c = wid + NW * j

        @pl.when(c < NCHUNK)
        def _():
            pltpu.sync_copy(dst_hbm.at[c], idx_v)
            pltpu.sync_copy(rows_v, acc_s.at[idx_v], add=True)

    plsc.subcore_barrier()

    @pl.loop(0, NZFULL)
    def _o(z):
        @pl.when(lax.rem(z, NS) == sid)
        def _():
            pltpu.sync_copy(acc_s.at[pl.ds(z * CH, CH), :], rows_v)
            pltpu.sync_copy(rows_v, cnt_hbm.at[cid, pl.ds(z * CH, CH), :])

    @pl.when(sid == NS - 1)
    def _orem():
        pltpu.sync_copy(acc_s.at[pl.ds(NZFULL * CH, NZREM), :],
                        rows_v.at[pl.ds(0, NZREM), :])
        pltpu.sync_copy(rows_v.at[pl.ds(0, NZREM), :],
                        cnt_hbm.at[cid, pl.ds(NZFULL * CH, NZREM), :])


def _final_body(s_ref, c_ref, w2_ref, b2_ref, w_ref, b_ref, o_ref):
    s = s_ref[0] + s_ref[1]
    cnt = c_ref[0, :, 0:1] + c_ref[1, :, 0:1]
    mean = s / jnp.maximum(cnt, 1.0)
    t = lax.dot_general(mean, w2_ref[...], (((1,), (1,)), ((), ())),
                        preferred_element_type=jnp.float32) + b2_ref[...]
    t = _ssp(t)
    o_ref[...] = lax.dot_general(t, w_ref[...], (((1,), (1,)), ((), ())),
                                 preferred_element_type=jnp.float32) + b_ref[...]


def _final(ssum, cnt, lin2_w, lin2_b, lin_w, lin_b):
    bn = 1000
    return pl.pallas_call(
        _final_body,
        grid=(N // bn,),
        in_specs=[pl.BlockSpec((NC, bn, F), lambda i: (0, i, 0)),
                  pl.BlockSpec((NC, bn, F), lambda i: (0, i, 0)),
                  pl.BlockSpec((H, F), lambda i: (0, 0)),
                  pl.BlockSpec((1, H), lambda i: (0, 0)),
                  pl.BlockSpec((H, H), lambda i: (0, 0)),
                  pl.BlockSpec((1, H), lambda i: (0, 0))],
        out_specs=pl.BlockSpec((bn, H), lambda i: (i, 0)),
        out_shape=jax.ShapeDtypeStruct((N, H), jnp.float32),
    )(ssum, cnt, lin2_w, lin2_b, lin_w, lin_b)


def kernel(x, edge_index, edge_weight, edge_attr, mlp_w1, mlp_b1, mlp_w2,
           mlp_b2, lin1_w, lin2_w, lin2_b, lin_w, lin_b):
    h = _lin1(x, lin1_w)
    src2 = edge_index[0].reshape(NCHUNK, CH)
    dst2 = edge_index[1].reshape(NCHUNK, CH)
    zrow = jnp.zeros((CH, F), jnp.float32)
    ones = jnp.ones((CH, F), jnp.float32)
    hsrc = _gather_k(h, src2)
    cnt = _count_k(dst2, zrow, ones)
    cc2 = _cutoff(edge_weight.reshape(NCHUNK, CH)).reshape(E, 1)
    msg = _edge_mlp(edge_attr, cc2,
                    hsrc.reshape(E, F), mlp_w1, mlp_b1.reshape(1, F),
                    mlp_w2, mlp_b2.reshape(1, F))
    ssum = _scatter_k(msg.reshape(NCHUNK, CH, F), dst2, zrow)
    return _final(ssum, cnt, lin2_w, lin2_b.reshape(1, H),
                  lin_w, lin_b.reshape(1, H))


# speedup vs baseline: 3.0254x; 1.1337x over previous
import functools
import math

import jax
import jax.numpy as jnp
from jax import lax
from jax.experimental import pallas as pl
from jax.experimental.pallas import tpu as pltpu
from jax.experimental.pallas import tpu_sc as plsc

N = 10000
E = 320000
H = 128
G = 50
F = 128
CUTOFF = 10.0
SHIFT = math.log(2.0)

NC = 2
NS = 16
NW = NC * NS
CH = 128
NCHUNK = E // CH
MAXJ = (NCHUNK + NW - 1) // NW

NZFULL = N // CH
NZREM = N - NZFULL * CH

NBUF = 4
LA = 2
NQUAD = (MAXJ + NBUF - 1) // NBUF

CHS = 64
NCHUNKS = E // CHS
MAXJS = (NCHUNKS + NW - 1) // NW
NQUADS = (MAXJS + NBUF - 1) // NBUF
NZFULL_S = N // CHS
NZREM_S = N - NZFULL_S * CHS
NBUFS = 3
NQUADS3 = (MAXJS + NBUFS - 1) // NBUFS


def _ssp(t):
    return jnp.log1p(jnp.exp(-jnp.abs(t))) + jnp.maximum(t, 0.0) - SHIFT


def _lin1_body(x_ref, w_ref, o_ref):
    o_ref[...] = lax.dot_general(
        x_ref[...], w_ref[...], (((1,), (1,)), ((), ())),
        preferred_element_type=jnp.float32)


def _lin1(x, w):
    bn = 1000
    return pl.pallas_call(
        _lin1_body,
        grid=(N // bn,),
        in_specs=[pl.BlockSpec((bn, H), lambda i: (i, 0)),
                  pl.BlockSpec((F, H), lambda i: (0, 0))],
        out_specs=pl.BlockSpec((bn, F), lambda i: (i, 0)),
        out_shape=jax.ShapeDtypeStruct((N, F), jnp.float32),
    )(x, w)


_MESH = plsc.VectorSubcoreMesh(
    core_axis_name="c", subcore_axis_name="s", num_cores=NC, num_subcores=NS)


def _preload_idx(src_hbm, idx_all, semI, wid, maxj=MAXJ, nchunk=NCHUNK):
    @pl.loop(0, maxj)
    def _pi(j):
        c = wid + NW * j

        @pl.when(c < nchunk)
        def _():
            pltpu.async_copy(src_hbm.at[c], idx_all.at[j], semI)

    @pl.loop(0, maxj)
    def _pw(j):
        c = wid + NW * j

        @pl.when(c < nchunk)
        def _():
            pltpu.make_async_copy(src_hbm.at[c], idx_all.at[j], semI).wait()


@functools.partial(
    pl.kernel,
    out_type=jax.ShapeDtypeStruct((NCHUNK, CH, F), jnp.float32),
    mesh=_MESH,
    scratch_types=[
        pltpu.VMEM((MAXJ, CH), jnp.int32),
        pltpu.VMEM((NBUF, CH, F), jnp.float32),
        pltpu.SemaphoreType.DMA,
        pltpu.SemaphoreType.DMA((NBUF,)),
        pltpu.SemaphoreType.DMA((NBUF,)),
    ],
)
def _gather_k(h_hbm, src_hbm, out_hbm, idx_all, rows_v, semI, semG, semW):
    wid = lax.axis_index("s") * NC + lax.axis_index("c")
    _preload_idx(src_hbm, idx_all, semI, wid)

    for j0 in range(LA):
        c0 = wid + NW * j0

        @pl.when(c0 < NCHUNK)
        def _(j0=j0):
            pltpu.async_copy(h_hbm.at[idx_all.at[j0]], rows_v.at[j0 % NBUF],
                             semG.at[j0 % NBUF])

    @pl.loop(0, NQUAD)
    def _main(u):
        base = u * NBUF
        for k in range(NBUF):
            j = base + k
            c = wid + NW * j
            jn = j + LA
            sn = (k + LA) % NBUF
            cn = wid + NW * jn

            @pl.when(jnp.logical_and(jn < MAXJ, cn < NCHUNK))
            def _(jn=jn, sn=sn):
                @pl.when(jn >= NBUF)
                def _():
                    cw = wid + NW * (jn - NBUF)
                    pltpu.make_async_copy(rows_v.at[sn], out_hbm.at[cw],
                                          semW.at[sn]).wait()
                pltpu.async_copy(h_hbm.at[idx_all.at[jn]], rows_v.at[sn],
                                 semG.at[sn])

            @pl.when(jnp.logical_and(j < MAXJ, c < NCHUNK))
            def _(j=j, k=k, c=c):
                pltpu.make_async_copy(h_hbm.at[idx_all.at[j]], rows_v.at[k],
                                      semG.at[k]).wait()
                pltpu.async_copy(rows_v.at[k], out_hbm.at[c], semW.at[k])

    for j in range(MAXJ - NBUF - 1, MAXJ):
        c = wid + NW * j
        cq = wid + NW * (j + NBUF)
        und = jnp.logical_or(j + NBUF >= MAXJ, cq >= NCHUNK)

        @pl.when(jnp.logical_and(c < NCHUNK, und))
        def _(j=j, c=c):
            pltpu.make_async_copy(rows_v.at[j % NBUF], out_hbm.at[c],
                                  semW.at[j % NBUF]).wait()


def _cw_body(ew_ref, o_ref):
    o_ref[...] = 0.5 * (jnp.cos(ew_ref[...] * (math.pi / CUTOFF)) + 1.0)


def _cutoff(ew2d):
    return pl.pallas_call(
        _cw_body,
        grid=(1,),
        in_specs=[pl.BlockSpec((NCHUNK, CH), lambda i: (0, 0))],
        out_specs=pl.BlockSpec((NCHUNK, CH), lambda i: (0, 0)),
        out_shape=jax.ShapeDtypeStruct((NCHUNK, CH), jnp.float32),
    )(ew2d)


def _edge_body(ea_ref, cc_ref, h_ref, w1_ref, b1_ref, w2_ref, b2_ref, o_ref):
    a = lax.dot_general(ea_ref[...], w1_ref[...], (((1,), (1,)), ((), ())),
                        preferred_element_type=jnp.float32) + b1_ref[...]
    a = _ssp(a)
    wf = lax.dot_general(a, w2_ref[...], (((1,), (1,)), ((), ())),
                         preferred_element_type=jnp.float32) + b2_ref[...]
    o_ref[...] = wf * cc_ref[...] * h_ref[...]


def _edge_mlp(edge_attr, cc2, hsrc, w1, b1, w2, b2):
    eb = 1280
    return pl.pallas_call(
        _edge_body,
        grid=(E // eb,),
        in_specs=[pl.BlockSpec((eb, G), lambda i: (i, 0)),
                  pl.BlockSpec((eb, 1), lambda i: (i, 0)),
                  pl.BlockSpec((eb, F), lambda i: (i, 0)),
                  pl.BlockSpec((F, G), lambda i: (0, 0)),
                  pl.BlockSpec((1, F), lambda i: (0, 0)),
                  pl.BlockSpec((F, F), lambda i: (0, 0)),
                  pl.BlockSpec((1, F), lambda i: (0, 0))],
        out_specs=pl.BlockSpec((eb, F), lambda i: (i, 0)),
        out_shape=jax.ShapeDtypeStruct((E, F), jnp.float32),
    )(edge_attr, cc2, hsrc, w1, b1, w2, b2)


@functools.partial(
    pl.kernel,
    out_type=jax.ShapeDtypeStruct((NC, N, F), jnp.float32),
    mesh=_MESH,
    scratch_types=[
        pltpu.VMEM((MAXJS, CHS), jnp.int32),
        pltpu.VMEM((NBUFS, CHS, F), jnp.float32),
        pltpu.VMEM_SHARED((N, F), jnp.float32),
        pltpu.SemaphoreType.DMA,
        pltpu.SemaphoreType.DMA((NBUFS,)),
        pltpu.SemaphoreType.DMA((NBUFS,)),
    ],
)
def _scatter_k(msg_hbm, dst_hbm, zrow_hbm, ssum_hbm, idx_all, rows_v, acc_s,
               semI, semM, semS):
    cid = lax.axis_index("c")
    sid = lax.axis_index("s")
    wid = sid * NC + cid

    _preload_idx(dst_hbm, idx_all, semI, wid, maxj=MAXJS, nchunk=NCHUNKS)

    pltpu.sync_copy(zrow_hbm, rows_v.at[0])

    @pl.loop(0, NZFULL_S)
    def _z(z):
        @pl.when(lax.rem(z, NS) == sid)
        def _():
            pltpu.sync_copy(rows_v.at[0], acc_s.at[pl.ds(z * CHS, CHS), :])

    @pl.when(sid == NS - 1)
    def _zrem():
        pltpu.sync_copy(rows_v.at[0, pl.ds(0, NZREM_S), :],
                        acc_s.at[pl.ds(NZFULL_S * CHS, NZREM_S), :])

    plsc.subcore_barrier()

    for j0 in range(LA):
        c0 = wid + NW * j0

        @pl.when(c0 < NCHUNKS)
        def _(j0=j0):
            pltpu.async_copy(msg_hbm.at[c0], rows_v.at[j0 % NBUFS],
                             semM.at[j0 % NBUFS])

    @pl.loop(0, NQUADS3)
    def _main(u):
        base = u * NBUFS
        for k in range(NBUFS):
            j = base + k
            c = wid + NW * j
            jn = j + LA
            sn = (k + LA) % NBUFS
            cn = wid + NW * jn

            @pl.when(jnp.logical_and(jn < MAXJS, cn < NCHUNKS))
            def _(jn=jn, sn=sn, cn=cn):
                @pl.when(jn >= NBUFS)
                def _():
                    jw = jn - NBUFS
                    pltpu.make_async_copy(rows_v.at[sn],
                                          acc_s.at[idx_all.at[jw]],
                                          semS.at[sn]).wait()
                pltpu.async_copy(msg_hbm.at[cn], rows_v.at[sn], semM.at[sn])

            @pl.when(jnp.logical_and(j < MAXJS, c < NCHUNKS))
            def _(j=j, k=k, c=c):
                pltpu.make_async_copy(msg_hbm.at[c], rows_v.at[k],
                                      semM.at[k]).wait()
                pltpu.async_copy(rows_v.at[k], acc_s.at[idx_all.at[j]],
                                 semS.at[k], add=True)

    for j in range(MAXJS - NBUFS - 1, MAXJS):
        c = wid + NW * j
        cq = wid + NW * (j + NBUFS)
        und = jnp.logical_or(j + NBUFS >= MAXJS, cq >= NCHUNKS)

        @pl.when(jnp.logical_and(c < NCHUNKS, und))
        def _(j=j):
            pltpu.make_async_copy(rows_v.at[j % NBUFS],
                                  acc_s.at[idx_all.at[j]],
                                  semS.at[j % NBUFS]).wait()

    plsc.subcore_barrier()

    @pl.loop(0, NZFULL_S)
    def _o(z):
        @pl.when(lax.rem(z, NS) == sid)
        def _():
            pltpu.sync_copy(acc_s.at[pl.ds(z * CHS, CHS), :], rows_v.at[0])
            pltpu.sync_copy(rows_v.at[0],
                            ssum_hbm.at[cid, pl.ds(z * CHS, CHS), :])

    @pl.when(sid == NS - 1)
    def _orem():
        pltpu.sync_copy(acc_s.at[pl.ds(NZFULL_S * CHS, NZREM_S), :],
                        rows_v.at[0, pl.ds(0, NZREM_S), :])
        pltpu.sync_copy(rows_v.at[0, pl.ds(0, NZREM_S), :],
                        ssum_hbm.at[cid, pl.ds(NZFULL_S * CHS, NZREM_S), :])


@functools.partial(
    pl.kernel,
    out_type=jax.ShapeDtypeStruct((NC, N, F), jnp.float32),
    mesh=_MESH,
    scratch_types=[
        pltpu.VMEM((MAXJ, CH), jnp.int32),
        pltpu.VMEM((CH, F), jnp.float32),
        pltpu.VMEM_SHARED((N, F), jnp.float32),
        pltpu.SemaphoreType.DMA,
        pltpu.SemaphoreType.DMA,
    ],
)
def _count_k(dst_hbm, zrow_hbm, ones_hbm, cnt_hbm, idx_all, rows_v, acc_s,
             semI, semA):
    cid = lax.axis_index("c")
    sid = lax.axis_index("s")
    wid = sid * NC + cid

    _preload_idx(dst_hbm, idx_all, semI, wid)

    pltpu.sync_copy(zrow_hbm, rows_v)

    @pl.loop(0, NZFULL)
    def _z(z):
        @pl.when(lax.rem(z, NS) == sid)
        def _():
            pltpu.sync_copy(rows_v, acc_s.at[pl.ds(z * CH, CH), :])

    @pl.when(sid == NS - 1)
    def _zrem():
        pltpu.sync_copy(rows_v.at[pl.ds(0, NZREM), :],
                        acc_s.at[pl.ds(NZFULL * CH, NZREM), :])

    pltpu.sync_copy(ones_hbm, rows_v)
    plsc.subcore_barrier()

    @pl.loop(0, MAXJ)
    def _chunks(j):
        c = wid + NW * j

        @pl.when(c < NCHUNK)
        def _():
            @pl.when(j >= NBUF)
            def _():
                jw = j - NBUF
                pltpu.make_async_copy(rows_v, acc_s.at[idx_all.at[jw]],
                                      semA).wait()
            pltpu.async_copy(rows_v, acc_s.at[idx_all.at[j]], semA, add=True)

    for j in range(MAXJ - NBUF - 1, MAXJ):
        c = wid + NW * j
        cq = wid + NW * (j + NBUF)
        und = jnp.logical_or(j + NBUF >= MAXJ, cq >= NCHUNK)

        @pl.when(jnp.logical_and(c < NCHUNK, und))
        def _(j=j):
            pltpu.make_async_copy(rows_v, acc_s.at[idx_all.at[j]],
                                  semA).wait()

    plsc.subcore_barrier()

    @pl.loop(0, NZFULL)
    def _o(z):
        @pl.when(lax.rem(z, NS) == sid)
        def _():
            pltpu.sync_copy(acc_s.at[pl.ds(z * CH, CH), :], rows_v)
            pltpu.sync_copy(rows_v, cnt_hbm.at[cid, pl.ds(z * CH, CH), :])

    @pl.when(sid == NS - 1)
    def _orem():
        pltpu.sync_copy(acc_s.at[pl.ds(NZFULL * CH, NZREM), :],
                        rows_v.at[pl.ds(0, NZREM), :])
        pltpu.sync_copy(rows_v.at[pl.ds(0, NZREM), :],
                        cnt_hbm.at[cid, pl.ds(NZFULL * CH, NZREM), :])


def _final_body(s_ref, c_ref, w2_ref, b2_ref, w_ref, b_ref, o_ref):
    s = s_ref[0] + s_ref[1]
    cnt = c_ref[0, :, 0:1] + c_ref[1, :, 0:1]
    mean = s / jnp.maximum(cnt, 1.0)
    t = lax.dot_general(mean, w2_ref[...], (((1,), (1,)), ((), ())),
                        preferred_element_type=jnp.float32) + b2_ref[...]
    t = _ssp(t)
    o_ref[...] = lax.dot_general(t, w_ref[...], (((1,), (1,)), ((), ())),
                                 preferred_element_type=jnp.float32) + b_ref[...]


def _final(ssum, cnt, lin2_w, lin2_b, lin_w, lin_b):
    bn = 1000
    return pl.pallas_call(
        _final_body,
        grid=(N // bn,),
        in_specs=[pl.BlockSpec((NC, bn, F), lambda i: (0, i, 0)),
                  pl.BlockSpec((NC, bn, F), lambda i: (0, i, 0)),
                  pl.BlockSpec((H, F), lambda i: (0, 0)),
                  pl.BlockSpec((1, H), lambda i: (0, 0)),
                  pl.BlockSpec((H, H), lambda i: (0, 0)),
                  pl.BlockSpec((1, H), lambda i: (0, 0))],
        out_specs=pl.BlockSpec((bn, H), lambda i: (i, 0)),
        out_shape=jax.ShapeDtypeStruct((N, H), jnp.float32),
    )(ssum, cnt, lin2_w, lin2_b, lin_w, lin_b)


def kernel(x, edge_index, edge_weight, edge_attr, mlp_w1, mlp_b1, mlp_w2,
           mlp_b2, lin1_w, lin2_w, lin2_b, lin_w, lin_b):
    h = _lin1(x, lin1_w)
    src2 = edge_index[0].reshape(NCHUNK, CH)
    dst2 = edge_index[1].reshape(NCHUNK, CH)
    zrow = jnp.zeros((CH, F), jnp.float32)
    ones = jnp.ones((CH, F), jnp.float32)
    hsrc = _gather_k(h, src2)
    cnt = _count_k(dst2, zrow, ones)
    cc2 = _cutoff(edge_weight.reshape(NCHUNK, CH)).reshape(E, 1)
    msg = _edge_mlp(edge_attr, cc2,
                    hsrc.reshape(E, F), mlp_w1, mlp_b1.reshape(1, F),
                    mlp_w2, mlp_b2.reshape(1, F))
    ssum = _scatter_k(msg.reshape(NCHUNKS, CHS, F),
                      edge_index[1].reshape(NCHUNKS, CHS),
                      jnp.zeros((CHS, F), jnp.float32))
    return _final(ssum, cnt, lin2_w, lin2_b.reshape(1, H),
                  lin_w, lin_b.reshape(1, H))


# speedup vs baseline: 3.0264x; 1.0003x over previous
import functools
import math

import jax
import jax.numpy as jnp
from jax import lax
from jax.experimental import pallas as pl
from jax.experimental.pallas import tpu as pltpu
from jax.experimental.pallas import tpu_sc as plsc

N = 10000
E = 320000
H = 128
G = 50
F = 128
CUTOFF = 10.0
SHIFT = math.log(2.0)

NC = 2
NS = 16
NW = NC * NS
CH = 128
NCHUNK = E // CH
MAXJ = (NCHUNK + NW - 1) // NW

NZFULL = N // CH
NZREM = N - NZFULL * CH

NBUF = 4
LA = 2
NQUAD = (MAXJ + NBUF - 1) // NBUF

CHS = 64
NCHUNKS = E // CHS
MAXJS = (NCHUNKS + NW - 1) // NW
NQUADS = (MAXJS + NBUF - 1) // NBUF
NZFULL_S = N // CHS
NZREM_S = N - NZFULL_S * CHS
NBUFS = 3
NQUADS3 = (MAXJS + NBUFS - 1) // NBUFS


def _ssp(t):
    return jnp.log1p(jnp.exp(-jnp.abs(t))) + jnp.maximum(t, 0.0) - SHIFT


def _lin1_body(x_ref, w_ref, o_ref):
    o_ref[...] = lax.dot_general(
        x_ref[...], w_ref[...], (((1,), (1,)), ((), ())),
        preferred_element_type=jnp.float32)


def _lin1(x, w):
    bn = 1000
    return pl.pallas_call(
        _lin1_body,
        grid=(N // bn,),
        in_specs=[pl.BlockSpec((bn, H), lambda i: (i, 0)),
                  pl.BlockSpec((F, H), lambda i: (0, 0))],
        out_specs=pl.BlockSpec((bn, F), lambda i: (i, 0)),
        out_shape=jax.ShapeDtypeStruct((N, F), jnp.float32),
    )(x, w)


_MESH = plsc.VectorSubcoreMesh(
    core_axis_name="c", subcore_axis_name="s", num_cores=NC, num_subcores=NS)


def _preload_idx(src_hbm, idx_all, semI, wid, maxj=MAXJ, nchunk=NCHUNK):
    @pl.loop(0, maxj)
    def _pi(j):
        c = wid + NW * j

        @pl.when(c < nchunk)
        def _():
            pltpu.async_copy(src_hbm.at[c], idx_all.at[j], semI)

    @pl.loop(0, maxj)
    def _pw(j):
        c = wid + NW * j

        @pl.when(c < nchunk)
        def _():
            pltpu.make_async_copy(src_hbm.at[c], idx_all.at[j], semI).wait()


@functools.partial(
    pl.kernel,
    out_type=jax.ShapeDtypeStruct((E, F), jnp.float32),
    mesh=_MESH,
    scratch_types=[
        pltpu.VMEM((MAXJ, CH), jnp.int32),
        pltpu.VMEM((NBUF, CH, F), jnp.float32),
        pltpu.SemaphoreType.DMA,
        pltpu.SemaphoreType.DMA((NBUF,)),
        pltpu.SemaphoreType.DMA((NBUF,)),
    ],
)
def _gather_k(h_hbm, src_hbm, out_hbm, idx_all, rows_v, semI, semG, semW):
    wid = lax.axis_index("s") * NC + lax.axis_index("c")
    _preload_idx(src_hbm, idx_all, semI, wid)

    for j0 in range(LA):
        c0 = wid + NW * j0

        @pl.when(c0 < NCHUNK)
        def _(j0=j0):
            pltpu.async_copy(h_hbm.at[idx_all.at[j0]], rows_v.at[j0 % NBUF],
                             semG.at[j0 % NBUF])

    @pl.loop(0, NQUAD)
    def _main(u):
        base = u * NBUF
        for k in range(NBUF):
            j = base + k
            c = wid + NW * j
            jn = j + LA
            sn = (k + LA) % NBUF
            cn = wid + NW * jn

            @pl.when(jnp.logical_and(jn < MAXJ, cn < NCHUNK))
            def _(jn=jn, sn=sn):
                @pl.when(jn >= NBUF)
                def _():
                    cw = wid + NW * (jn - NBUF)
                    pltpu.make_async_copy(
                        rows_v.at[sn],
                        out_hbm.at[pl.ds(cw * CH, CH), :],
                        semW.at[sn]).wait()
                pltpu.async_copy(h_hbm.at[idx_all.at[jn]], rows_v.at[sn],
                                 semG.at[sn])

            @pl.when(jnp.logical_and(j < MAXJ, c < NCHUNK))
            def _(j=j, k=k, c=c):
                pltpu.make_async_copy(h_hbm.at[idx_all.at[j]], rows_v.at[k],
                                      semG.at[k]).wait()
                pltpu.async_copy(rows_v.at[k],
                                 out_hbm.at[pl.ds(c * CH, CH), :], semW.at[k])

    for j in range(MAXJ - NBUF - 1, MAXJ):
        c = wid + NW * j
        cq = wid + NW * (j + NBUF)
        und = jnp.logical_or(j + NBUF >= MAXJ, cq >= NCHUNK)

        @pl.when(jnp.logical_and(c < NCHUNK, und))
        def _(j=j, c=c):
            pltpu.make_async_copy(rows_v.at[j % NBUF],
                                  out_hbm.at[pl.ds(c * CH, CH), :],
                                  semW.at[j % NBUF]).wait()


def _cw_body(ew_ref, o_ref):
    o_ref[...] = 0.5 * (jnp.cos(ew_ref[...] * (math.pi / CUTOFF)) + 1.0)


def _cutoff(ew2d):
    return pl.pallas_call(
        _cw_body,
        grid=(1,),
        in_specs=[pl.BlockSpec((NCHUNK, CH), lambda i: (0, 0))],
        out_specs=pl.BlockSpec((NCHUNK, CH), lambda i: (0, 0)),
        out_shape=jax.ShapeDtypeStruct((NCHUNK, CH), jnp.float32),
    )(ew2d)


def _edge_body(ea_ref, cc_ref, h_ref, w1_ref, b1_ref, w2_ref, b2_ref, o_ref):
    a = lax.dot_general(ea_ref[...], w1_ref[...], (((1,), (1,)), ((), ())),
                        preferred_element_type=jnp.float32) + b1_ref[...]
    a = _ssp(a)
    wf = lax.dot_general(a, w2_ref[...], (((1,), (1,)), ((), ())),
                         preferred_element_type=jnp.float32) + b2_ref[...]
    o_ref[...] = wf * cc_ref[...] * h_ref[...]


def _edge_mlp(edge_attr, cc2, hsrc, w1, b1, w2, b2):
    eb = 1280
    return pl.pallas_call(
        _edge_body,
        grid=(E // eb,),
        in_specs=[pl.BlockSpec((eb, G), lambda i: (i, 0)),
                  pl.BlockSpec((eb, 1), lambda i: (i, 0)),
                  pl.BlockSpec((eb, F), lambda i: (i, 0)),
                  pl.BlockSpec((F, G), lambda i: (0, 0)),
                  pl.BlockSpec((1, F), lambda i: (0, 0)),
                  pl.BlockSpec((F, F), lambda i: (0, 0)),
                  pl.BlockSpec((1, F), lambda i: (0, 0))],
        out_specs=pl.BlockSpec((eb, F), lambda i: (i, 0)),
        out_shape=jax.ShapeDtypeStruct((E, F), jnp.float32),
    )(edge_attr, cc2, hsrc, w1, b1, w2, b2)


@functools.partial(
    pl.kernel,
    out_type=jax.ShapeDtypeStruct((NC, N, F), jnp.float32),
    mesh=_MESH,
    scratch_types=[
        pltpu.VMEM((MAXJS, CHS), jnp.int32),
        pltpu.VMEM((NBUFS, CHS, F), jnp.float32),
        pltpu.VMEM_SHARED((N, F), jnp.float32),
        pltpu.SemaphoreType.DMA,
        pltpu.SemaphoreType.DMA((NBUFS,)),
        pltpu.SemaphoreType.DMA((NBUFS,)),
    ],
)
def _scatter_k(msg_hbm, dst_hbm, zrow_hbm, ssum_hbm, idx_all, rows_v, acc_s,
               semI, semM, semS):
    cid = lax.axis_index("c")
    sid = lax.axis_index("s")
    wid = sid * NC + cid

    _preload_idx(dst_hbm, idx_all, semI, wid, maxj=MAXJS, nchunk=NCHUNKS)

    pltpu.sync_copy(zrow_hbm, rows_v.at[0])

    @pl.loop(0, NZFULL_S)
    def _z(z):
        @pl.when(lax.rem(z, NS) == sid)
        def _():
            pltpu.sync_copy(rows_v.at[0], acc_s.at[pl.ds(z * CHS, CHS), :])

    @pl.when(sid == NS - 1)
    def _zrem():
        pltpu.sync_copy(rows_v.at[0, pl.ds(0, NZREM_S), :],
                        acc_s.at[pl.ds(NZFULL_S * CHS, NZREM_S), :])

    plsc.subcore_barrier()

    for j0 in range(LA):
        c0 = wid + NW * j0

        @pl.when(c0 < NCHUNKS)
        def _(j0=j0):
            pltpu.async_copy(msg_hbm.at[pl.ds(c0 * CHS, CHS), :],
                             rows_v.at[j0 % NBUFS], semM.at[j0 % NBUFS])

    @pl.loop(0, NQUADS3)
    def _main(u):
        base = u * NBUFS
        for k in range(NBUFS):
            j = base + k
            c = wid + NW * j
            jn = j + LA
            sn = (k + LA) % NBUFS
            cn = wid + NW * jn

            @pl.when(jnp.logical_and(jn < MAXJS, cn < NCHUNKS))
            def _(jn=jn, sn=sn, cn=cn):
                @pl.when(jn >= NBUFS)
                def _():
                    jw = jn - NBUFS
                    pltpu.make_async_copy(rows_v.at[sn],
                                          acc_s.at[idx_all.at[jw]],
                                          semS.at[sn]).wait()
                pltpu.async_copy(msg_hbm.at[pl.ds(cn * CHS, CHS), :],
                                 rows_v.at[sn], semM.at[sn])

            @pl.when(jnp.logical_and(j < MAXJS, c < NCHUNKS))
            def _(j=j, k=k, c=c):
                pltpu.make_async_copy(msg_hbm.at[pl.ds(c * CHS, CHS), :],
                                      rows_v.at[k], semM.at[k]).wait()
                pltpu.async_copy(rows_v.at[k], acc_s.at[idx_all.at[j]],
                                 semS.at[k], add=True)

    for j in range(MAXJS - NBUFS - 1, MAXJS):
        c = wid + NW * j
        cq = wid + NW * (j + NBUFS)
        und = jnp.logical_or(j + NBUFS >= MAXJS, cq >= NCHUNKS)

        @pl.when(jnp.logical_and(c < NCHUNKS, und))
        def _(j=j):
            pltpu.make_async_copy(rows_v.at[j % NBUFS],
                                  acc_s.at[idx_all.at[j]],
                                  semS.at[j % NBUFS]).wait()

    plsc.subcore_barrier()

    @pl.loop(0, NZFULL_S)
    def _o(z):
        @pl.when(lax.rem(z, NS) == sid)
        def _():
            pltpu.sync_copy(acc_s.at[pl.ds(z * CHS, CHS), :], rows_v.at[0])
            pltpu.sync_copy(rows_v.at[0],
                            ssum_hbm.at[cid, pl.ds(z * CHS, CHS), :])

    @pl.when(sid == NS - 1)
    def _orem():
        pltpu.sync_copy(acc_s.at[pl.ds(NZFULL_S * CHS, NZREM_S), :],
                        rows_v.at[0, pl.ds(0, NZREM_S), :])
        pltpu.sync_copy(rows_v.at[0, pl.ds(0, NZREM_S), :],
                        ssum_hbm.at[cid, pl.ds(NZFULL_S * CHS, NZREM_S), :])


@functools.partial(
    pl.kernel,
    out_type=jax.ShapeDtypeStruct((NC, N, F), jnp.float32),
    mesh=_MESH,
    scratch_types=[
        pltpu.VMEM((MAXJ, CH), jnp.int32),
        pltpu.VMEM((CH, F), jnp.float32),
        pltpu.VMEM_SHARED((N, F), jnp.float32),
        pltpu.SemaphoreType.DMA,
        pltpu.SemaphoreType.DMA,
    ],
)
def _count_k(dst_hbm, zrow_hbm, ones_hbm, cnt_hbm, idx_all, rows_v, acc_s,
             semI, semA):
    cid = lax.axis_index("c")
    sid = lax.axis_index("s")
    wid = sid * NC + cid

    _preload_idx(dst_hbm, idx_all, semI, wid)

    pltpu.sync_copy(zrow_hbm, rows_v)

    @pl.loop(0, NZFULL)
    def _z(z):
        @pl.when(lax.rem(z, NS) == sid)
        def _():
            pltpu.sync_copy(rows_v, acc_s.at[pl.ds(z * CH, CH), :])

    @pl.when(sid == NS - 1)
    def _zrem():
        pltpu.sync_copy(rows_v.at[pl.ds(0, NZREM), :],
                        acc_s.at[pl.ds(NZFULL * CH, NZREM), :])

    pltpu.sync_copy(ones_hbm, rows_v)
    plsc.subcore_barrier()

    @pl.loop(0, MAXJ)
    def _chunks(j):
        c = wid + NW * j

        @pl.when(c < NCHUNK)
        def _():
            @pl.when(j >= NBUF)
            def _():
                jw = j - NBUF
                pltpu.make_async_copy(rows_v, acc_s.at[idx_all.at[jw]],
                                      semA).wait()
            pltpu.async_copy(rows_v, acc_s.at[idx_all.at[j]], semA, add=True)

    for j in range(MAXJ - NBUF - 1, MAXJ):
        c = wid + NW * j
        cq = wid + NW * (j + NBUF)
        und = jnp.logical_or(j + NBUF >= MAXJ, cq >= NCHUNK)

        @pl.when(jnp.logical_and(c < NCHUNK, und))
        def _(j=j):
            pltpu.make_async_copy(rows_v, acc_s.at[idx_all.at[j]],
                                  semA).wait()

    plsc.subcore_barrier()

    @pl.loop(0, NZFULL)
    def _o(z):
        @pl.when(lax.rem(z, NS) == sid)
        def _():
            pltpu.sync_copy(acc_s.at[pl.ds(z * CH, CH), :], rows_v)
            pltpu.sync_copy(rows_v, cnt_hbm.at[cid, pl.ds(z * CH, CH), :])

    @pl.when(sid == NS - 1)
    def _orem():
        pltpu.sync_copy(acc_s.at[pl.ds(NZFULL * CH, NZREM), :],
                        rows_v.at[pl.ds(0, NZREM), :])
        pltpu.sync_copy(rows_v.at[pl.ds(0, NZREM), :],
                        cnt_hbm.at[cid, pl.ds(NZFULL * CH, NZREM), :])


def _final_body(s_ref, c_ref, w2_ref, b2_ref, w_ref, b_ref, o_ref):
    s = s_ref[0] + s_ref[1]
    cnt = c_ref[0, :, 0:1] + c_ref[1, :, 0:1]
    mean = s / jnp.maximum(cnt, 1.0)
    t = lax.dot_general(mean, w2_ref[...], (((1,), (1,)), ((), ())),
                        preferred_element_type=jnp.float32) + b2_ref[...]
    t = _ssp(t)
    o_ref[...] = lax.dot_general(t, w_ref[...], (((1,), (1,)), ((), ())),
                                 preferred_element_type=jnp.float32) + b_ref[...]


def _final(ssum, cnt, lin2_w, lin2_b, lin_w, lin_b):
    bn = 1000
    return pl.pallas_call(
        _final_body,
        grid=(N // bn,),
        in_specs=[pl.BlockSpec((NC, bn, F), lambda i: (0, i, 0)),
                  pl.BlockSpec((NC, bn, F), lambda i: (0, i, 0)),
                  pl.BlockSpec((H, F), lambda i: (0, 0)),
                  pl.BlockSpec((1, H), lambda i: (0, 0)),
                  pl.BlockSpec((H, H), lambda i: (0, 0)),
                  pl.BlockSpec((1, H), lambda i: (0, 0))],
        out_specs=pl.BlockSpec((bn, H), lambda i: (i, 0)),
        out_shape=jax.ShapeDtypeStruct((N, H), jnp.float32),
    )(ssum, cnt, lin2_w, lin2_b, lin_w, lin_b)


def kernel(x, edge_index, edge_weight, edge_attr, mlp_w1, mlp_b1, mlp_w2,
           mlp_b2, lin1_w, lin2_w, lin2_b, lin_w, lin_b):
    h = _lin1(x, lin1_w)
    src2 = edge_index[0].reshape(NCHUNK, CH)
    dst2 = edge_index[1].reshape(NCHUNK, CH)
    zrow = jnp.zeros((CH, F), jnp.float32)
    ones = jnp.ones((CH, F), jnp.float32)
    hsrc = _gather_k(h, src2)
    cnt = _count_k(dst2, zrow, ones)
    cc2 = _cutoff(edge_weight.reshape(NCHUNK, CH)).reshape(E, 1)
    msg = _edge_mlp(edge_attr, cc2,
                    hsrc, mlp_w1, mlp_b1.reshape(1, F),
                    mlp_w2, mlp_b2.reshape(1, F))
    ssum = _scatter_k(msg, edge_index[1].reshape(NCHUNKS, CHS),
                      jnp.zeros((CHS, F), jnp.float32))
    return _final(ssum, cnt, lin2_w, lin2_b.reshape(1, H),
                  lin_w, lin_b.reshape(1, H))


# speedup vs baseline: 3.8351x; 1.2672x over previous
import functools
import math

import jax
import jax.numpy as jnp
from jax import lax
from jax.experimental import pallas as pl
from jax.experimental.pallas import tpu as pltpu
from jax.experimental.pallas import tpu_sc as plsc

N = 10000
E = 320000
H = 128
G = 50
F = 128
CUTOFF = 10.0
SHIFT = math.log(2.0)

NC = 2
NS = 16
NW = NC * NS
CH = 128
NCHUNK = E // CH
MAXJ = (NCHUNK + NW - 1) // NW

NZFULL = N // CH
NZREM = N - NZFULL * CH

NBUF = 4
LA = 2
NQUAD = (MAXJ + NBUF - 1) // NBUF

CHS = 64
NCHUNKS = E // CHS
MAXJS = (NCHUNKS + NW - 1) // NW
NQUADS = (MAXJS + NBUF - 1) // NBUF
NZFULL_S = N // CHS
NZREM_S = N - NZFULL_S * CHS
NBUFS = 3
NQUADS3 = (MAXJS + NBUFS - 1) // NBUFS


def _ssp(t):
    return jnp.log1p(jnp.exp(-jnp.abs(t))) + jnp.maximum(t, 0.0) - SHIFT


def _lin1_body(x_ref, w_ref, o_ref):
    o_ref[...] = lax.dot_general(
        x_ref[...], w_ref[...], (((1,), (1,)), ((), ())),
        preferred_element_type=jnp.float32)


def _lin1(x, w):
    bn = 1000
    return pl.pallas_call(
        _lin1_body,
        grid=(N // bn,),
        in_specs=[pl.BlockSpec((bn, H), lambda i: (i, 0)),
                  pl.BlockSpec((F, H), lambda i: (0, 0))],
        out_specs=pl.BlockSpec((bn, F), lambda i: (i, 0)),
        out_shape=jax.ShapeDtypeStruct((N, F), jnp.float32),
    )(x, w)


_MESH = plsc.VectorSubcoreMesh(
    core_axis_name="c", subcore_axis_name="s", num_cores=NC, num_subcores=NS)


def _preload_idx(src_hbm, idx_all, semI, wid, maxj=MAXJ, nchunk=NCHUNK):
    @pl.loop(0, maxj)
    def _pi(j):
        c = wid + NW * j

        @pl.when(c < nchunk)
        def _():
            pltpu.async_copy(src_hbm.at[c], idx_all.at[j], semI)

    @pl.loop(0, maxj)
    def _pw(j):
        c = wid + NW * j

        @pl.when(c < nchunk)
        def _():
            pltpu.make_async_copy(src_hbm.at[c], idx_all.at[j], semI).wait()


@functools.partial(
    pl.kernel,
    out_type=jax.ShapeDtypeStruct((E, F), jnp.float32),
    mesh=_MESH,
    scratch_types=[
        pltpu.VMEM((MAXJ, CH), jnp.int32),
        pltpu.VMEM((NBUF, CH, F), jnp.float32),
        pltpu.SemaphoreType.DMA,
        pltpu.SemaphoreType.DMA((NBUF,)),
        pltpu.SemaphoreType.DMA((NBUF,)),
    ],
)
def _gather_k(h_hbm, src_hbm, out_hbm, idx_all, rows_v, semI, semG, semW):
    wid = lax.axis_index("s") * NC + lax.axis_index("c")
    _preload_idx(src_hbm, idx_all, semI, wid)

    for j0 in range(LA):
        c0 = wid + NW * j0

        @pl.when(c0 < NCHUNK)
        def _(j0=j0):
            pltpu.async_copy(h_hbm.at[idx_all.at[j0]], rows_v.at[j0 % NBUF],
                             semG.at[j0 % NBUF])

    @pl.loop(0, NQUAD)
    def _main(u):
        base = u * NBUF
        for k in range(NBUF):
            j = base + k
            c = wid + NW * j
            jn = j + LA
            sn = (k + LA) % NBUF
            cn = wid + NW * jn

            @pl.when(jnp.logical_and(jn < MAXJ, cn < NCHUNK))
            def _(jn=jn, sn=sn):
                @pl.when(jn >= NBUF)
                def _():
                    cw = wid + NW * (jn - NBUF)
                    pltpu.make_async_copy(
                        rows_v.at[sn],
                        out_hbm.at[pl.ds(cw * CH, CH), :],
                        semW.at[sn]).wait()
                pltpu.async_copy(h_hbm.at[idx_all.at[jn]], rows_v.at[sn],
                                 semG.at[sn])

            @pl.when(jnp.logical_and(j < MAXJ, c < NCHUNK))
            def _(j=j, k=k, c=c):
                pltpu.make_async_copy(h_hbm.at[idx_all.at[j]], rows_v.at[k],
                                      semG.at[k]).wait()
                pltpu.async_copy(rows_v.at[k],
                                 out_hbm.at[pl.ds(c * CH, CH), :], semW.at[k])

    for j in range(MAXJ - NBUF - 1, MAXJ):
        c = wid + NW * j
        cq = wid + NW * (j + NBUF)
        und = jnp.logical_or(j + NBUF >= MAXJ, cq >= NCHUNK)

        @pl.when(jnp.logical_and(c < NCHUNK, und))
        def _(j=j, c=c):
            pltpu.make_async_copy(rows_v.at[j % NBUF],
                                  out_hbm.at[pl.ds(c * CH, CH), :],
                                  semW.at[j % NBUF]).wait()


def _cw_body(ew_ref, o_ref):
    o_ref[...] = 0.5 * (jnp.cos(ew_ref[...] * (math.pi / CUTOFF)) + 1.0)


def _cutoff(ew2d):
    return pl.pallas_call(
        _cw_body,
        grid=(1,),
        in_specs=[pl.BlockSpec((NCHUNK, CH), lambda i: (0, 0))],
        out_specs=pl.BlockSpec((NCHUNK, CH), lambda i: (0, 0)),
        out_shape=jax.ShapeDtypeStruct((NCHUNK, CH), jnp.float32),
    )(ew2d)


def _edge_body(ea_ref, ew_ref, h_ref, w1_ref, b1_ref, w2_ref, b2_ref, o_ref):
    a = lax.dot_general(ea_ref[...], w1_ref[...], (((1,), (1,)), ((), ())),
                        preferred_element_type=jnp.float32) + b1_ref[...]
    a = _ssp(a)
    wf = lax.dot_general(a, w2_ref[...], (((1,), (1,)), ((), ())),
                         preferred_element_type=jnp.float32) + b2_ref[...]
    cc = 0.5 * (jnp.cos(ew_ref[...] * (math.pi / CUTOFF)) + 1.0)
    cct = cc.T
    wfh = wf * h_ref[...]
    for t in range(wfh.shape[0] // 128):
        o_ref[pl.ds(t * 128, 128), :] = (
            wfh[t * 128:(t + 1) * 128, :] * cct[:, t:t + 1])


def _edge_mlp(edge_attr, ew2d, hsrc, w1, b1, w2, b2):
    eb = 2048
    return pl.pallas_call(
        _edge_body,
        grid=((E + eb - 1) // eb,),
        in_specs=[pl.BlockSpec((eb, G), lambda i: (i, 0)),
                  pl.BlockSpec((eb // CH, CH), lambda i: (i, 0)),
                  pl.BlockSpec((eb, F), lambda i: (i, 0)),
                  pl.BlockSpec((F, G), lambda i: (0, 0)),
                  pl.BlockSpec((1, F), lambda i: (0, 0)),
                  pl.BlockSpec((F, F), lambda i: (0, 0)),
                  pl.BlockSpec((1, F), lambda i: (0, 0))],
        out_specs=pl.BlockSpec((eb, F), lambda i: (i, 0)),
        out_shape=jax.ShapeDtypeStruct((E, F), jnp.float32),
    )(edge_attr, ew2d, hsrc, w1, b1, w2, b2)


@functools.partial(
    pl.kernel,
    out_type=jax.ShapeDtypeStruct((NC, N, F), jnp.float32),
    mesh=_MESH,
    scratch_types=[
        pltpu.VMEM((MAXJS, CHS), jnp.int32),
        pltpu.VMEM((NBUFS, CHS, F), jnp.float32),
        pltpu.VMEM_SHARED((N, F), jnp.float32),
        pltpu.SemaphoreType.DMA,
        pltpu.SemaphoreType.DMA((NBUFS,)),
        pltpu.SemaphoreType.DMA((NBUFS,)),
    ],
)
def _scatter_k(msg_hbm, dst_hbm, zrow_hbm, ssum_hbm, idx_all, rows_v, acc_s,
               semI, semM, semS):
    cid = lax.axis_index("c")
    sid = lax.axis_index("s")
    wid = sid * NC + cid

    _preload_idx(dst_hbm, idx_all, semI, wid, maxj=MAXJS, nchunk=NCHUNKS)

    pltpu.sync_copy(zrow_hbm, rows_v.at[0])

    @pl.loop(0, NZFULL_S)
    def _z(z):
        @pl.when(lax.rem(z, NS) == sid)
        def _():
            pltpu.sync_copy(rows_v.at[0], acc_s.at[pl.ds(z * CHS, CHS), :])

    @pl.when(sid == NS - 1)
    def _zrem():
        pltpu.sync_copy(rows_v.at[0, pl.ds(0, NZREM_S), :],
                        acc_s.at[pl.ds(NZFULL_S * CHS, NZREM_S), :])

    plsc.subcore_barrier()

    for j0 in range(LA):
        c0 = wid + NW * j0

        @pl.when(c0 < NCHUNKS)
        def _(j0=j0):
            pltpu.async_copy(msg_hbm.at[pl.ds(c0 * CHS, CHS), :],
                             rows_v.at[j0 % NBUFS], semM.at[j0 % NBUFS])

    @pl.loop(0, NQUADS3)
    def _main(u):
        base = u * NBUFS
        for k in range(NBUFS):
            j = base + k
            c = wid + NW * j
            jn = j + LA
            sn = (k + LA) % NBUFS
            cn = wid + NW * jn

            @pl.when(jnp.logical_and(jn < MAXJS, cn < NCHUNKS))
            def _(jn=jn, sn=sn, cn=cn):
                @pl.when(jn >= NBUFS)
                def _():
                    jw = jn - NBUFS
                    pltpu.make_async_copy(rows_v.at[sn],
                                          acc_s.at[idx_all.at[jw]],
                                          semS.at[sn]).wait()
                pltpu.async_copy(msg_hbm.at[pl.ds(cn * CHS, CHS), :],
                                 rows_v.at[sn], semM.at[sn])

            @pl.when(jnp.logical_and(j < MAXJS, c < NCHUNKS))
            def _(j=j, k=k, c=c):
                pltpu.make_async_copy(msg_hbm.at[pl.ds(c * CHS, CHS), :],
                                      rows_v.at[k], semM.at[k]).wait()
                pltpu.async_copy(rows_v.at[k], acc_s.at[idx_all.at[j]],
                                 semS.at[k], add=True)

    for j in range(MAXJS - NBUFS - 1, MAXJS):
        c = wid + NW * j
        cq = wid + NW * (j + NBUFS)
        und = jnp.logical_or(j + NBUFS >= MAXJS, cq >= NCHUNKS)

        @pl.when(jnp.logical_and(c < NCHUNKS, und))
        def _(j=j):
            pltpu.make_async_copy(rows_v.at[j % NBUFS],
                                  acc_s.at[idx_all.at[j]],
                                  semS.at[j % NBUFS]).wait()

    plsc.subcore_barrier()

    @pl.loop(0, NZFULL_S)
    def _o(z):
        @pl.when(lax.rem(z, NS) == sid)
        def _():
            pltpu.sync_copy(acc_s.at[pl.ds(z * CHS, CHS), :], rows_v.at[0])
            pltpu.sync_copy(rows_v.at[0],
                            ssum_hbm.at[cid, pl.ds(z * CHS, CHS), :])

    @pl.when(sid == NS - 1)
    def _orem():
        pltpu.sync_copy(acc_s.at[pl.ds(NZFULL_S * CHS, NZREM_S), :],
                        rows_v.at[0, pl.ds(0, NZREM_S), :])
        pltpu.sync_copy(rows_v.at[0, pl.ds(0, NZREM_S), :],
                        ssum_hbm.at[cid, pl.ds(NZFULL_S * CHS, NZREM_S), :])


@functools.partial(
    pl.kernel,
    out_type=jax.ShapeDtypeStruct((NC, N, F), jnp.float32),
    mesh=_MESH,
    scratch_types=[
        pltpu.VMEM((MAXJ, CH), jnp.int32),
        pltpu.VMEM((CH, F), jnp.float32),
        pltpu.VMEM_SHARED((N, F), jnp.float32),
        pltpu.SemaphoreType.DMA,
        pltpu.SemaphoreType.DMA,
    ],
)
def _count_k(dst_hbm, zrow_hbm, ones_hbm, cnt_hbm, idx_all, rows_v, acc_s,
             semI, semA):
    cid = lax.axis_index("c")
    sid = lax.axis_index("s")
    wid = sid * NC + cid

    _preload_idx(dst_hbm, idx_all, semI, wid)

    pltpu.sync_copy(zrow_hbm, rows_v)

    @pl.loop(0, NZFULL)
    def _z(z):
        @pl.when(lax.rem(z, NS) == sid)
        def _():
            pltpu.sync_copy(rows_v, acc_s.at[pl.ds(z * CH, CH), :])

    @pl.when(sid == NS - 1)
    def _zrem():
        pltpu.sync_copy(rows_v.at[pl.ds(0, NZREM), :],
                        acc_s.at[pl.ds(NZFULL * CH, NZREM), :])

    pltpu.sync_copy(ones_hbm, rows_v)
    plsc.subcore_barrier()

    @pl.loop(0, MAXJ)
    def _chunks(j):
        c = wid + NW * j

        @pl.when(c < NCHUNK)
        def _():
            @pl.when(j >= NBUF)
            def _():
                jw = j - NBUF
                pltpu.make_async_copy(rows_v, acc_s.at[idx_all.at[jw]],
                                      semA).wait()
            pltpu.async_copy(rows_v, acc_s.at[idx_all.at[j]], semA, add=True)

    for j in range(MAXJ - NBUF - 1, MAXJ):
        c = wid + NW * j
        cq = wid + NW * (j + NBUF)
        und = jnp.logical_or(j + NBUF >= MAXJ, cq >= NCHUNK)

        @pl.when(jnp.logical_and(c < NCHUNK, und))
        def _(j=j):
            pltpu.make_async_copy(rows_v, acc_s.at[idx_all.at[j]],
                                  semA).wait()

    plsc.subcore_barrier()

    @pl.loop(0, NZFULL)
    def _o(z):
        @pl.when(lax.rem(z, NS) == sid)
        def _():
            pltpu.sync_copy(acc_s.at[pl.ds(z * CH, CH), :], rows_v)
            pltpu.sync_copy(rows_v, cnt_hbm.at[cid, pl.ds(z * CH, CH), :])

    @pl.when(sid == NS - 1)
    def _orem():
        pltpu.sync_copy(acc_s.at[pl.ds(NZFULL * CH, NZREM), :],
                        rows_v.at[pl.ds(0, NZREM), :])
        pltpu.sync_copy(rows_v.at[pl.ds(0, NZREM), :],
                        cnt_hbm.at[cid, pl.ds(NZFULL * CH, NZREM), :])


def _final_body(s_ref, c_ref, w2_ref, b2_ref, w_ref, b_ref, o_ref):
    s = s_ref[0] + s_ref[1]
    cnt = c_ref[0, :, 0:1] + c_ref[1, :, 0:1]
    mean = s / jnp.maximum(cnt, 1.0)
    t = lax.dot_general(mean, w2_ref[...], (((1,), (1,)), ((), ())),
                        preferred_element_type=jnp.float32) + b2_ref[...]
    t = _ssp(t)
    o_ref[...] = lax.dot_general(t, w_ref[...], (((1,), (1,)), ((), ())),
                                 preferred_element_type=jnp.float32) + b_ref[...]


def _final(ssum, cnt, lin2_w, lin2_b, lin_w, lin_b):
    bn = 1000
    return pl.pallas_call(
        _final_body,
        grid=(N // bn,),
        in_specs=[pl.BlockSpec((NC, bn, F), lambda i: (0, i, 0)),
                  pl.BlockSpec((NC, bn, F), lambda i: (0, i, 0)),
                  pl.BlockSpec((H, F), lambda i: (0, 0)),
                  pl.BlockSpec((1, H), lambda i: (0, 0)),
                  pl.BlockSpec((H, H), lambda i: (0, 0)),
                  pl.BlockSpec((1, H), lambda i: (0, 0))],
        out_specs=pl.BlockSpec((bn, H), lambda i: (i, 0)),
        out_shape=jax.ShapeDtypeStruct((N, H), jnp.float32),
    )(ssum, cnt, lin2_w, lin2_b, lin_w, lin_b)


def kernel(x, edge_index, edge_weight, edge_attr, mlp_w1, mlp_b1, mlp_w2,
           mlp_b2, lin1_w, lin2_w, lin2_b, lin_w, lin_b):
    h = _lin1(x, lin1_w)
    src2 = edge_index[0].reshape(NCHUNK, CH)
    dst2 = edge_index[1].reshape(NCHUNK, CH)
    zrow = jnp.zeros((CH, F), jnp.float32)
    ones = jnp.ones((CH, F), jnp.float32)
    hsrc = _gather_k(h, src2)
    cnt = _count_k(dst2, zrow, ones)
    msg = _edge_mlp(edge_attr, edge_weight.reshape(NCHUNK, CH),
                    hsrc, mlp_w1, mlp_b1.reshape(1, F),
                    mlp_w2, mlp_b2.reshape(1, F))
    ssum = _scatter_k(msg, edge_index[1].reshape(NCHUNKS, CHS),
                      jnp.zeros((CHS, F), jnp.float32))
    return _final(ssum, cnt, lin2_w, lin2_b.reshape(1, H),
                  lin_w, lin_b.reshape(1, H))
